# Initial kernel scaffold; baseline (speedup 1.0000x reference)
import functools

import jax
import jax.numpy as jnp
from jax import lax
from jax.experimental import pallas as pl
from jax.experimental.pallas import tpu as pltpu
from jax.experimental.pallas import tpu_sc as plsc

NC = 2
NS = 16
L = 16
NW = NC * NS
C = 128
D = 128


def _spmm_sc(src, rows, cols, vals):
    n = src.shape[0]
    ept = rows.shape[0] // NW
    n_chunks = ept // C
    acc_rows = ((n + NS * C - 1) // (NS * C)) * (NS * C)
    zchunks = acc_rows // NS // C
    drain = n // NS

    mesh = plsc.VectorSubcoreMesh(core_axis_name="c", subcore_axis_name="s")

    @functools.partial(
        pl.kernel,
        out_type=jax.ShapeDtypeStruct((NC * n, D), jnp.float32),
        mesh=mesh,
        scratch_types=[
            pltpu.VMEM_SHARED((acc_rows, D), jnp.float32),
            pltpu.VMEM((C, D), jnp.float32),
            pltpu.VMEM((C,), jnp.int32),
            pltpu.VMEM((C,), jnp.int32),
            pltpu.VMEM((C,), jnp.float32),
        ],
    )
    def k(src_hbm, rows_hbm, cols_hbm, vals_hbm, out_hbm,
          acc, gbuf, col_v, row_v, val_v):
        c = lax.axis_index("c")
        s = lax.axis_index("s")
        wid = c * NS + s

        zero16 = jnp.zeros((L,), jnp.float32)

        def zrow(i, carry):
            for j in range(D // L):
                gbuf[i, pl.ds(j * L, L)] = zero16
            return carry

        lax.fori_loop(0, C, zrow, 0)
        zbase = s * (acc_rows // NS)
        for z in range(zchunks):
            pltpu.sync_copy(gbuf, acc.at[pl.ds(zbase + z * C, C)])
        plsc.subcore_barrier()

        ebase = wid * ept

        def chunk(i, carry):
            base = ebase + i * C
            pltpu.sync_copy(cols_hbm.at[pl.ds(base, C)], col_v)
            pltpu.sync_copy(rows_hbm.at[pl.ds(base, C)], row_v)
            pltpu.sync_copy(vals_hbm.at[pl.ds(base, C)], val_v)
            pltpu.sync_copy(src_hbm.at[col_v], gbuf)

            def srow(r, carry2):
                sc = plsc.load_gather(val_v, [jnp.broadcast_to(r, (L,))])
                for j in range(D // L):
                    sl = pl.ds(j * L, L)
                    gbuf[r, sl] = gbuf[r, sl] * sc
                return carry2

            lax.fori_loop(0, C, srow, 0)
            pltpu.sync_copy(gbuf, acc.at[row_v], add=True)
            return carry

        lax.fori_loop(0, n_chunks, chunk, 0)
        plsc.subcore_barrier()

        dbase = s * drain
        pltpu.sync_copy(acc.at[pl.ds(dbase, drain)],
                        out_hbm.at[pl.ds(c * n + dbase, drain)])

    return k(src, rows, cols, vals)


def _combine(partials, prev, alpha, beta, n):
    bn = 1000
    nb = n // bn

    def body(a_ref, b_ref, p_ref, o_ref):
        o_ref[...] = (alpha * (a_ref[...] + b_ref[...])
                      + beta * p_ref[...])

    return pl.pallas_call(
        body,
        grid=(nb,),
        in_specs=[
            pl.BlockSpec((bn, D), lambda i: (i, 0)),
            pl.BlockSpec((bn, D), lambda i: (i + nb, 0)),
            pl.BlockSpec((bn, D), lambda i: (i, 0)),
        ],
        out_specs=pl.BlockSpec((bn, D), lambda i: (i, 0)),
        out_shape=jax.ShapeDtypeStruct((n, D), jnp.float32),
    )(partials, partials, prev)


def _final(xi, t1, t2, p3, w, b, n):
    bn = 1000
    nb = n // bn

    def body(x_ref, t1_ref, t2_ref, pa_ref, pb_ref, w_ref, b_ref, o_ref):
        t1b = t1_ref[...]
        acc = jnp.dot(x_ref[...], w_ref[0], preferred_element_type=jnp.float32)
        acc += jnp.dot(t1b, w_ref[1], preferred_element_type=jnp.float32)
        acc += jnp.dot(t2_ref[...], w_ref[2], preferred_element_type=jnp.float32)
        t3b = 2.0 * (pa_ref[...] + pb_ref[...]) - t1b
        acc += jnp.dot(t3b, w_ref[3], preferred_element_type=jnp.float32)
        o_ref[...] = jnp.maximum(acc + b_ref[...], 0.0)

    return pl.pallas_call(
        body,
        grid=(nb,),
        in_specs=[
            pl.BlockSpec((bn, D), lambda i: (i, 0)),
            pl.BlockSpec((bn, D), lambda i: (i, 0)),
            pl.BlockSpec((bn, D), lambda i: (i, 0)),
            pl.BlockSpec((bn, D), lambda i: (i, 0)),
            pl.BlockSpec((bn, D), lambda i: (i + nb, 0)),
            pl.BlockSpec((4, D, D), lambda i: (0, 0, 0)),
            pl.BlockSpec((1, D), lambda i: (0, 0)),
        ],
        out_specs=pl.BlockSpec((bn, D), lambda i: (i, 0)),
        out_shape=jax.ShapeDtypeStruct((n, D), jnp.float32),
    )(xi, t1, t2, p3, p3, w, b)


def kernel(x, lap_indices, lap_values, W, b):
    n = x.shape[1]
    e = lap_indices.shape[1]
    rows = lap_indices[0].astype(jnp.int32)
    cols = lap_indices[1].astype(jnp.int32)
    vals = lap_values.astype(jnp.float32)
    ep = ((e + NW * C - 1) // (NW * C)) * (NW * C)
    pad = ep - e
    if pad:
        rows = jnp.pad(rows, (0, pad))
        cols = jnp.pad(cols, (0, pad))
        vals = jnp.pad(vals, (0, pad))
    b2 = b.reshape(1, D).astype(jnp.float32)
    w = W.astype(jnp.float32)

    outs = []
    for i in range(x.shape[0]):
        xi = x[i]
        p1 = _spmm_sc(xi, rows, cols, vals)
        t1 = _combine(p1, xi, 1.0, 0.0, n)
        p2 = _spmm_sc(t1, rows, cols, vals)
        t2 = _combine(p2, xi, 2.0, -1.0, n)
        p3 = _spmm_sc(t2, rows, cols, vals)
        outs.append(_final(xi, t1, t2, p3, w, b2, n))
    return jnp.stack(outs, axis=0)


# speedup vs baseline: 2.9113x; 2.9113x over previous
import functools

import jax
import jax.numpy as jnp
from jax import lax
from jax.experimental import pallas as pl
from jax.experimental.pallas import tpu as pltpu
from jax.experimental.pallas import tpu_sc as plsc

NC = 2
NS = 16
L = 16
NW = NC * NS
C = 128
D = 128


def _spmm_sc(src, rows, cols, vals):
    n = src.shape[0]
    ept = rows.shape[0] // NW
    n_chunks = ept // C
    acc_rows = ((n + NS * C - 1) // (NS * C)) * (NS * C)
    zchunks = acc_rows // NS // C
    drain = acc_rows // NS

    mesh = plsc.VectorSubcoreMesh(core_axis_name="c", subcore_axis_name="s")

    @functools.partial(
        pl.kernel,
        out_type=jax.ShapeDtypeStruct((NC * acc_rows, D), jnp.float32),
        mesh=mesh,
        scratch_types=[
            pltpu.VMEM_SHARED((acc_rows, D), jnp.float32),
            pltpu.VMEM((C, D), jnp.float32),
            pltpu.VMEM((C,), jnp.int32),
            pltpu.VMEM((C,), jnp.int32),
            pltpu.VMEM((C,), jnp.float32),
        ],
    )
    def k(src_hbm, rows_hbm, cols_hbm, vals_hbm, out_hbm,
          acc, gbuf, col_v, row_v, val_v):
        c = lax.axis_index("c")
        s = lax.axis_index("s")
        wid = c * NS + s

        zero16 = jnp.zeros((L,), jnp.float32)

        def zrow(i, carry):
            for j in range(D // L):
                gbuf[i, pl.ds(j * L, L)] = zero16
            return carry

        lax.fori_loop(0, C, zrow, 0)
        zbase = s * (acc_rows // NS)
        for z in range(zchunks):
            pltpu.sync_copy(gbuf, acc.at[pl.ds(zbase + z * C, C)])
        plsc.subcore_barrier()

        ebase = wid * ept

        def chunk(i, carry):
            base = ebase + i * C
            pltpu.sync_copy(cols_hbm.at[pl.ds(base, C)], col_v)
            pltpu.sync_copy(rows_hbm.at[pl.ds(base, C)], row_v)
            pltpu.sync_copy(vals_hbm.at[pl.ds(base, C)], val_v)
            pltpu.sync_copy(src_hbm.at[col_v], gbuf)

            def sgroup(g, carry2):
                v16 = val_v[pl.ds(g * L, L)]
                for rloc in range(L):
                    sc = v16[rloc]
                    r = g * L + rloc
                    for j in range(D // L):
                        sl = pl.ds(j * L, L)
                        gbuf[r, sl] = gbuf[r, sl] * sc
                return carry2

            lax.fori_loop(0, C // L, sgroup, 0)
            pltpu.sync_copy(gbuf, acc.at[row_v], add=True)
            return carry

        lax.fori_loop(0, n_chunks, chunk, 0)
        plsc.subcore_barrier()

        dbase = s * drain
        pltpu.sync_copy(acc.at[pl.ds(dbase, drain)],
                        out_hbm.at[pl.ds(c * acc_rows + dbase, drain)])

    return k(src, rows, cols, vals)


def _combine(partials, prev, alpha, beta, n, acc_rows):
    bn = 80
    nb = n // bn
    off = acc_rows // bn

    def body(a_ref, b_ref, p_ref, o_ref):
        o_ref[...] = (alpha * (a_ref[...] + b_ref[...])
                      + beta * p_ref[...])

    return pl.pallas_call(
        body,
        grid=(nb,),
        in_specs=[
            pl.BlockSpec((bn, D), lambda i: (i, 0)),
            pl.BlockSpec((bn, D), lambda i: (i + off, 0)),
            pl.BlockSpec((bn, D), lambda i: (i, 0)),
        ],
        out_specs=pl.BlockSpec((bn, D), lambda i: (i, 0)),
        out_shape=jax.ShapeDtypeStruct((n, D), jnp.float32),
    )(partials, partials, prev)


def _final(xi, t1, t2, p3, w, b, n, acc_rows):
    bn = 80
    nb = n // bn
    off = acc_rows // bn

    def body(x_ref, t1_ref, t2_ref, pa_ref, pb_ref, w_ref, b_ref, o_ref):
        t1b = t1_ref[...]
        acc = jnp.dot(x_ref[...], w_ref[0], preferred_element_type=jnp.float32)
        acc += jnp.dot(t1b, w_ref[1], preferred_element_type=jnp.float32)
        acc += jnp.dot(t2_ref[...], w_ref[2], preferred_element_type=jnp.float32)
        t3b = 2.0 * (pa_ref[...] + pb_ref[...]) - t1b
        acc += jnp.dot(t3b, w_ref[3], preferred_element_type=jnp.float32)
        o_ref[...] = jnp.maximum(acc + b_ref[...], 0.0)

    return pl.pallas_call(
        body,
        grid=(nb,),
        in_specs=[
            pl.BlockSpec((bn, D), lambda i: (i, 0)),
            pl.BlockSpec((bn, D), lambda i: (i, 0)),
            pl.BlockSpec((bn, D), lambda i: (i, 0)),
            pl.BlockSpec((bn, D), lambda i: (i, 0)),
            pl.BlockSpec((bn, D), lambda i: (i + off, 0)),
            pl.BlockSpec((4, D, D), lambda i: (0, 0, 0)),
            pl.BlockSpec((1, D), lambda i: (0, 0)),
        ],
        out_specs=pl.BlockSpec((bn, D), lambda i: (i, 0)),
        out_shape=jax.ShapeDtypeStruct((n, D), jnp.float32),
    )(xi, t1, t2, p3, p3, w, b)


def kernel(x, lap_indices, lap_values, W, b):
    n = x.shape[1]
    e = lap_indices.shape[1]
    rows = lap_indices[0].astype(jnp.int32)
    cols = lap_indices[1].astype(jnp.int32)
    vals = lap_values.astype(jnp.float32)
    ep = ((e + NW * C - 1) // (NW * C)) * (NW * C)
    pad = ep - e
    if pad:
        rows = jnp.pad(rows, (0, pad))
        cols = jnp.pad(cols, (0, pad))
        vals = jnp.pad(vals, (0, pad))
    b2 = b.reshape(1, D).astype(jnp.float32)
    w = W.astype(jnp.float32)

    acc_rows = ((n + NS * C - 1) // (NS * C)) * (NS * C)
    outs = []
    for i in range(x.shape[0]):
        xi = x[i]
        p1 = _spmm_sc(xi, rows, cols, vals)
        t1 = _combine(p1, xi, 1.0, 0.0, n, acc_rows)
        p2 = _spmm_sc(t1, rows, cols, vals)
        t2 = _combine(p2, xi, 2.0, -1.0, n, acc_rows)
        p3 = _spmm_sc(t2, rows, cols, vals)
        outs.append(_final(xi, t1, t2, p3, w, b2, n, acc_rows))
    return jnp.stack(outs, axis=0)
